# baseline (speedup 1.0000x reference)
import functools

import jax
import jax.numpy as jnp
from jax import lax
from jax.experimental import pallas as pl
from jax.experimental.pallas import tpu as pltpu
from jax.experimental.pallas import tpu_sc as plsc

NUM_CODES = 512
CODE_DIM = 32
BATCH = 65536
TILE = 4096
CHUNKS = 2
CHUNK = BATCH // CHUNKS

_SC_CORES = 2
_SC_SUBCORES = 16


def _argmin_kernel(zt_ref, cb_ref, idx_ref, loss_ref):
    i = pl.program_id(0)
    zt = zt_ref[...]
    cb = cb_ref[...]

    z2 = jnp.sum(zt * zt, axis=0, keepdims=True)
    cb2 = jnp.sum(cb * cb, axis=1, keepdims=True)
    dotm2 = jax.lax.dot_general(
        -2.0 * cb, zt,
        dimension_numbers=(((1,), (0,)), ((), ())),
        preferred_element_type=jnp.float32,
    )
    d = (z2 + dotm2) + cb2
    m = jnp.min(d, axis=0, keepdims=True)
    code_iota = jax.lax.broadcasted_iota(jnp.int32, (NUM_CODES, TILE), 0)
    idx = jnp.min(jnp.where(d == m, code_iota, NUM_CODES),
                  axis=0, keepdims=True)
    idx_ref[...] = idx.reshape(TILE)

    part = jnp.sum(m)
    acc = jnp.where(i == 0, jnp.zeros((1, 1), jnp.float32), loss_ref[...])
    loss_ref[...] = acc + part


def _tc_argmin(zt, codebook, chunk):
    tiles = CHUNK // TILE
    idx, loss_sum = pl.pallas_call(
        _argmin_kernel,
        grid=(tiles,),
        in_specs=[
            pl.BlockSpec((CODE_DIM, TILE),
                         lambda i, c=chunk, t=tiles: (0, c * t + i)),
            pl.BlockSpec((NUM_CODES, CODE_DIM), lambda i: (0, 0)),
        ],
        out_specs=[
            pl.BlockSpec((TILE,), lambda i: (i,)),
            pl.BlockSpec((1, 1), lambda i: (0, 0)),
        ],
        out_shape=[
            jax.ShapeDtypeStruct((CHUNK,), jnp.int32),
            jax.ShapeDtypeStruct((1, 1), jnp.float32),
        ],
    )(zt, codebook)
    return idx, loss_sum


def _sc_gather(codebook, idx):
    nw = _SC_CORES * _SC_SUBCORES
    b_per_w = CHUNK // nw
    mesh = plsc.VectorSubcoreMesh(core_axis_name="c", subcore_axis_name="s")

    @functools.partial(
        pl.kernel, mesh=mesh,
        compiler_params=pltpu.CompilerParams(use_tc_tiling_on_sc=False),
        out_type=jax.ShapeDtypeStruct((CHUNK, CODE_DIM), jnp.float32),
        scratch_types=[
            pltpu.VMEM((b_per_w,), jnp.int32),
            pltpu.VMEM((b_per_w, CODE_DIM), jnp.float32),
            pltpu.SemaphoreType.DMA,
        ],
    )
    def gather(table_hbm, idx_hbm, out_hbm, idx_v, rows_v, sem):
        wid = lax.axis_index("s") * _SC_CORES + lax.axis_index("c")
        base = wid * b_per_w
        pltpu.sync_copy(idx_hbm.at[pl.ds(base, b_per_w)], idx_v)
        pltpu.async_copy(table_hbm.at[idx_v], rows_v, sem).wait()
        pltpu.sync_copy(rows_v, out_hbm.at[pl.ds(base, b_per_w)])

    return gather(codebook, idx)


@jax.jit
def kernel(z_e, codebook):
    zt = z_e.T
    idxs, zqs, loss_sums = [], [], []
    for c in range(CHUNKS):
        idx_c, loss_c = _tc_argmin(zt, codebook, c)
        zqs.append(_sc_gather(codebook, idx_c))
        idxs.append(idx_c)
        loss_sums.append(loss_c[0, 0])
    zq = jnp.concatenate(zqs, axis=0)
    idx = jnp.concatenate(idxs, axis=0)
    commitment_loss = sum(loss_sums) / (BATCH * CODE_DIM)
    return (zq, idx, commitment_loss)


# speedup vs baseline: 1.0310x; 1.0310x over previous
import functools

import jax
import jax.numpy as jnp
from jax import lax
from jax.experimental import pallas as pl
from jax.experimental.pallas import tpu as pltpu
from jax.experimental.pallas import tpu_sc as plsc

NUM_CODES = 512
CODE_DIM = 32
BATCH = 65536
TILE = 4096
CHUNKS = 1
CHUNK = BATCH // CHUNKS

_SC_CORES = 2
_SC_SUBCORES = 16


def _argmin_kernel(zt_ref, cb_ref, idx_ref, loss_ref):
    i = pl.program_id(0)
    zt = zt_ref[...]
    cb = cb_ref[...]

    z2 = jnp.sum(zt * zt, axis=0, keepdims=True)
    cb2 = jnp.sum(cb * cb, axis=1, keepdims=True)
    dotm2 = jax.lax.dot_general(
        -2.0 * cb, zt,
        dimension_numbers=(((1,), (0,)), ((), ())),
        preferred_element_type=jnp.float32,
    )
    d = (z2 + dotm2) + cb2
    m = jnp.min(d, axis=0, keepdims=True)
    code_iota = jax.lax.broadcasted_iota(jnp.int32, (NUM_CODES, TILE), 0)
    idx = jnp.min(jnp.where(d == m, code_iota, NUM_CODES),
                  axis=0, keepdims=True)
    idx_ref[...] = idx.reshape(TILE)

    part = jnp.sum(m)
    acc = jnp.where(i == 0, jnp.zeros((1, 1), jnp.float32), loss_ref[...])
    loss_ref[...] = acc + part


def _tc_argmin(zt, codebook, chunk):
    tiles = CHUNK // TILE
    idx, loss_sum = pl.pallas_call(
        _argmin_kernel,
        grid=(tiles,),
        in_specs=[
            pl.BlockSpec((CODE_DIM, TILE),
                         lambda i, c=chunk, t=tiles: (0, c * t + i)),
            pl.BlockSpec((NUM_CODES, CODE_DIM), lambda i: (0, 0)),
        ],
        out_specs=[
            pl.BlockSpec((TILE,), lambda i: (i,)),
            pl.BlockSpec((1, 1), lambda i: (0, 0)),
        ],
        out_shape=[
            jax.ShapeDtypeStruct((CHUNK,), jnp.int32),
            jax.ShapeDtypeStruct((1, 1), jnp.float32),
        ],
    )(zt, codebook)
    return idx, loss_sum


def _sc_gather(codebook, idx):
    nw = _SC_CORES * _SC_SUBCORES
    b_per_w = CHUNK // nw
    mesh = plsc.VectorSubcoreMesh(core_axis_name="c", subcore_axis_name="s")

    @functools.partial(
        pl.kernel, mesh=mesh,
        compiler_params=pltpu.CompilerParams(use_tc_tiling_on_sc=False),
        out_type=jax.ShapeDtypeStruct((CHUNK, CODE_DIM), jnp.float32),
        scratch_types=[
            pltpu.VMEM((b_per_w,), jnp.int32),
            pltpu.VMEM((b_per_w, CODE_DIM), jnp.float32),
            pltpu.SemaphoreType.DMA,
        ],
    )
    def gather(table_hbm, idx_hbm, out_hbm, idx_v, rows_v, sem):
        wid = lax.axis_index("s") * _SC_CORES + lax.axis_index("c")
        base = wid * b_per_w
        pltpu.sync_copy(idx_hbm.at[pl.ds(base, b_per_w)], idx_v)
        pltpu.async_copy(table_hbm.at[idx_v], rows_v, sem).wait()
        pltpu.sync_copy(rows_v, out_hbm.at[pl.ds(base, b_per_w)])

    return gather(codebook, idx)


_TILE_ROWS = CODE_DIM // 8
_LANE_TILES = BATCH // 128


def _sc_gather_tiled(codebook_flat, idx):
    nw = _SC_CORES * _SC_SUBCORES
    b_per_w = BATCH // nw
    groups = b_per_w // 16
    tiles_w = b_per_w // 128
    mesh = plsc.VectorSubcoreMesh(core_axis_name="c", subcore_axis_name="s")

    @functools.partial(
        pl.kernel, mesh=mesh,
        compiler_params=pltpu.CompilerParams(use_tc_tiling_on_sc=False,
                                             needs_layout_passes=False),
        out_type=jax.ShapeDtypeStruct(
            (_TILE_ROWS, BATCH * CODE_DIM // _TILE_ROWS), jnp.float32),
        scratch_types=[
            pltpu.VMEM((b_per_w,), jnp.int32),
            pltpu.VMEM((NUM_CODES * CODE_DIM,), jnp.float32),
            pltpu.VMEM((b_per_w * CODE_DIM,), jnp.float32),
        ],
    )
    def gather(table_hbm, idx_hbm, out_hbm, idx_v, cb_v, buf_v):
        wid = lax.axis_index("s") * _SC_CORES + lax.axis_index("c")
        base = wid * b_per_w
        pltpu.sync_copy(idx_hbm.at[pl.ds(base, b_per_w)], idx_v)
        pltpu.sync_copy(table_hbm, cb_v)

        def body(j, _):
            iv = idx_v[pl.ds(j * 16, 16)]
            iv32 = iv * CODE_DIM
            tl = j // 8
            off = (j % 8) * 16
            col_base = tl * 1024 + off
            for k in range(CODE_DIM):
                vals = plsc.load_gather(cb_v, [iv32 + k])
                rr, r = k // 8, k % 8
                buf_v[pl.ds(col_base + rr * (tiles_w * 1024) + r * 128, 16)] \
                    = vals
            return _

        lax.fori_loop(0, groups, body, 0)

        per_row = tiles_w * 1024
        for rr in range(_TILE_ROWS):
            pltpu.sync_copy(
                buf_v.at[pl.ds(rr * per_row, per_row)],
                out_hbm.at[rr, pl.ds(wid * per_row, per_row)])

    out = gather(codebook_flat, idx)
    return (out.reshape(_TILE_ROWS, _LANE_TILES, 8, 128)
            .transpose(1, 3, 0, 2).reshape(BATCH, CODE_DIM))


@jax.jit
def kernel(z_e, codebook):
    zt = z_e.T
    idxs, loss_sums = [], []
    for c in range(CHUNKS):
        idx_c, loss_c = _tc_argmin(zt, codebook, c)
        idxs.append(idx_c)
        loss_sums.append(loss_c[0, 0])
    idx = jnp.concatenate(idxs, axis=0) if CHUNKS > 1 else idxs[0]
    zq = _sc_gather_tiled(codebook.reshape(NUM_CODES * CODE_DIM), idx)
    commitment_loss = sum(loss_sums) / (BATCH * CODE_DIM)
    return (zq, idx, commitment_loss)


# speedup vs baseline: 1.2493x; 1.2117x over previous
import functools

import jax
import jax.numpy as jnp
from jax import lax
from jax.experimental import pallas as pl
from jax.experimental.pallas import tpu as pltpu
from jax.experimental.pallas import tpu_sc as plsc

NUM_CODES = 512
CODE_DIM = 32
BATCH = 65536
TILE = 4096
CHUNKS = 1
CHUNK = BATCH // CHUNKS

_SC_CORES = 2
_SC_SUBCORES = 16


def _argmin_kernel(zt_ref, cb_ref, idx_ref, loss_ref):
    i = pl.program_id(0)
    zt = zt_ref[...]
    cb = cb_ref[...]

    z2 = jnp.sum(zt * zt, axis=0, keepdims=True)
    cb2 = jnp.sum(cb * cb, axis=1, keepdims=True)
    dotm2 = jax.lax.dot_general(
        -2.0 * cb, zt,
        dimension_numbers=(((1,), (0,)), ((), ())),
        preferred_element_type=jnp.float32,
    )
    d = (z2 + dotm2) + cb2
    m = jnp.min(d, axis=0, keepdims=True)
    code_iota = jax.lax.broadcasted_iota(jnp.int32, (NUM_CODES, TILE), 0)
    idx = jnp.min(jnp.where(d == m, code_iota, NUM_CODES),
                  axis=0, keepdims=True)
    idx_ref[...] = idx.reshape(TILE)

    part = jnp.sum(m)
    acc = jnp.where(i == 0, jnp.zeros((1, 1), jnp.float32), loss_ref[...])
    loss_ref[...] = acc + part


def _tc_argmin(zt, codebook, chunk):
    tiles = CHUNK // TILE
    idx, loss_sum = pl.pallas_call(
        _argmin_kernel,
        grid=(tiles,),
        in_specs=[
            pl.BlockSpec((CODE_DIM, TILE),
                         lambda i, c=chunk, t=tiles: (0, c * t + i)),
            pl.BlockSpec((NUM_CODES, CODE_DIM), lambda i: (0, 0)),
        ],
        out_specs=[
            pl.BlockSpec((TILE,), lambda i: (i,)),
            pl.BlockSpec((1, 1), lambda i: (0, 0)),
        ],
        out_shape=[
            jax.ShapeDtypeStruct((CHUNK,), jnp.int32),
            jax.ShapeDtypeStruct((1, 1), jnp.float32),
        ],
    )(zt, codebook)
    return idx, loss_sum


def _sc_gather(codebook, idx):
    nw = _SC_CORES * _SC_SUBCORES
    b_per_w = CHUNK // nw
    mesh = plsc.VectorSubcoreMesh(core_axis_name="c", subcore_axis_name="s")

    @functools.partial(
        pl.kernel, mesh=mesh,
        compiler_params=pltpu.CompilerParams(use_tc_tiling_on_sc=False),
        out_type=jax.ShapeDtypeStruct((CHUNK, CODE_DIM), jnp.float32),
        scratch_types=[
            pltpu.VMEM((b_per_w,), jnp.int32),
            pltpu.VMEM((b_per_w, CODE_DIM), jnp.float32),
            pltpu.SemaphoreType.DMA,
        ],
    )
    def gather(table_hbm, idx_hbm, out_hbm, idx_v, rows_v, sem):
        wid = lax.axis_index("s") * _SC_CORES + lax.axis_index("c")
        base = wid * b_per_w
        pltpu.sync_copy(idx_hbm.at[pl.ds(base, b_per_w)], idx_v)
        pltpu.async_copy(table_hbm.at[idx_v], rows_v, sem).wait()
        pltpu.sync_copy(rows_v, out_hbm.at[pl.ds(base, b_per_w)])

    return gather(codebook, idx)


_TILE_ROWS = CODE_DIM // 8
_LANE_TILES = BATCH // 128


def _sc_gather_tiled(codebook_flat, idx):
    nw = _SC_CORES * _SC_SUBCORES
    b_per_w = BATCH // nw
    groups = b_per_w // 16
    tiles_w = b_per_w // 128
    mesh = plsc.VectorSubcoreMesh(core_axis_name="c", subcore_axis_name="s")

    @functools.partial(
        pl.kernel, mesh=mesh,
        compiler_params=pltpu.CompilerParams(use_tc_tiling_on_sc=False,
                                             needs_layout_passes=False),
        out_type=jax.ShapeDtypeStruct(
            (_TILE_ROWS, BATCH * CODE_DIM // _TILE_ROWS), jnp.float32),
        scratch_types=[
            pltpu.VMEM((b_per_w,), jnp.int32),
            pltpu.VMEM((NUM_CODES * CODE_DIM,), jnp.float32),
            pltpu.VMEM((b_per_w * CODE_DIM,), jnp.float32),
        ],
    )
    def gather(table_hbm, idx_hbm, out_hbm, idx_v, cb_v, buf_v):
        wid = lax.axis_index("s") * _SC_CORES + lax.axis_index("c")
        base = wid * b_per_w
        pltpu.sync_copy(idx_hbm.at[pl.ds(base, b_per_w)], idx_v)
        pltpu.sync_copy(table_hbm, cb_v)

        @plsc.parallel_loop(0, groups, 1, unroll=2)
        def body(j):
            iv = idx_v[pl.ds(j * 16, 16)]
            iv32 = iv * CODE_DIM
            tl = j // 8
            off = (j % 8) * 16
            col_base = tl * 1024 + off
            for k in range(CODE_DIM):
                vals = plsc.load_gather(cb_v, [iv32 + k])
                rr, r = k // 8, k % 8
                buf_v[pl.ds(col_base + rr * (tiles_w * 1024) + r * 128, 16)] \
                    = vals

        per_row = tiles_w * 1024
        for rr in range(_TILE_ROWS):
            pltpu.sync_copy(
                buf_v.at[pl.ds(rr * per_row, per_row)],
                out_hbm.at[rr, pl.ds(wid * per_row, per_row)])

    out = gather(codebook_flat, idx)
    return (out.reshape(_TILE_ROWS, _LANE_TILES, 8, 128)
            .transpose(1, 3, 0, 2).reshape(BATCH, CODE_DIM))


@jax.jit
def kernel(z_e, codebook):
    zt = z_e.T
    idxs, loss_sums = [], []
    for c in range(CHUNKS):
        idx_c, loss_c = _tc_argmin(zt, codebook, c)
        idxs.append(idx_c)
        loss_sums.append(loss_c[0, 0])
    idx = jnp.concatenate(idxs, axis=0) if CHUNKS > 1 else idxs[0]
    zq = _sc_gather_tiled(codebook.reshape(NUM_CODES * CODE_DIM), idx)
    commitment_loss = sum(loss_sums) / (BATCH * CODE_DIM)
    return (zq, idx, commitment_loss)
